# Initial kernel scaffold; baseline (speedup 1.0000x reference)
import jax
import jax.numpy as jnp
from jax.experimental import pallas as pl


def kernel(cons_embedding, vals_embedding, cons_embedding_0, vals_embedding_0, v2c_edge_index, c2v_edge_index, v2c_edge_attr, c2v_edge_attr, cons_batch, vals_batch, W_e1, W1, b1, W_e2, W2, b2):
    raise NotImplementedError("write your pallas kernel here")


# speedup vs baseline: 2.4321x; 2.4321x over previous
import functools

import jax
import jax.numpy as jnp
from jax import lax
from jax.experimental import pallas as pl
from jax.experimental.pallas import tpu as pltpu
from jax.experimental.pallas import tpu_sc as plsc

N = 10000
E = 320000
D = 128
DE = 4
NC = 2
NS = 16
NW = NC * NS
EPW = E // NW
K = 80
NCHUNK = EPW // K
RPT = 624
TAIL = N - NS * RPT
ZR = 208


def _bias_body(a1_ref, a2_ref, w1_ref, w2_ref, p1_ref, p2_ref):
    p1_ref[...] = jnp.dot(a1_ref[...], w1_ref[...],
                          preferred_element_type=jnp.float32)
    p2_ref[...] = jnp.dot(a2_ref[...], w2_ref[...],
                          preferred_element_type=jnp.float32)


def _edge_bias(attr1, attr2, we1, we2):
    BE = 8000
    return pl.pallas_call(
        _bias_body,
        grid=(E // BE,),
        in_specs=[
            pl.BlockSpec((BE, DE), lambda i: (i, 0)),
            pl.BlockSpec((BE, DE), lambda i: (i, 0)),
            pl.BlockSpec((DE, D), lambda i: (0, 0)),
            pl.BlockSpec((DE, D), lambda i: (0, 0)),
        ],
        out_specs=[
            pl.BlockSpec((BE, D), lambda i: (i, 0)),
            pl.BlockSpec((BE, D), lambda i: (i, 0)),
        ],
        out_shape=[
            jax.ShapeDtypeStruct((E, D), jnp.float32),
            jax.ShapeDtypeStruct((E, D), jnp.float32),
        ],
    )(attr1, attr2, we1, we2)


def _update_body(x_ref, agg_ref, w_ref, b_ref, o_ref):
    s = x_ref[...] + agg_ref[0] + agg_ref[1]
    y = jnp.dot(s, w_ref[...], preferred_element_type=jnp.float32)
    o_ref[...] = jnp.maximum(y + b_ref[...], 0.0)


def _update(x, agg, w, b2d):
    R = 2000
    return pl.pallas_call(
        _update_body,
        grid=(N // R,),
        in_specs=[
            pl.BlockSpec((R, D), lambda i: (i, 0)),
            pl.BlockSpec((NC, R, D), lambda i: (0, i, 0)),
            pl.BlockSpec((D, D), lambda i: (0, 0)),
            pl.BlockSpec((1, D), lambda i: (0, 0)),
        ],
        out_specs=pl.BlockSpec((R, D), lambda i: (i, 0)),
        out_shape=jax.ShapeDtypeStruct((N, D), jnp.float32),
    )(x, agg, w, b2d)


def _sc_agg(xsrc, sidx, didx, p):
    mesh = plsc.VectorSubcoreMesh(core_axis_name="c", subcore_axis_name="s")

    @functools.partial(
        pl.kernel,
        mesh=mesh,
        out_type=jax.ShapeDtypeStruct((NC, N, D), jnp.float32),
        scratch_types=[
            pltpu.VMEM((K,), jnp.int32),
            pltpu.VMEM((K,), jnp.int32),
            pltpu.VMEM((K, D), jnp.float32),
            pltpu.VMEM((K, D), jnp.float32),
            pltpu.VMEM((ZR, D), jnp.float32),
            pltpu.VMEM_SHARED((N, D), jnp.float32),
            pltpu.SemaphoreType.DMA,
        ],
    )
    def k(xsrc_hbm, sidx_hbm, didx_hbm, p_hbm, out_hbm,
          sidx_v, didx_v, rows_v, p_v, zero_v, acc_sh, sem):
        cid = lax.axis_index("c")
        sid = lax.axis_index("s")
        zvec = jnp.zeros((16,), jnp.float32)

        def zrow(r, carry):
            for g in range(D // 16):
                zero_v[r, pl.ds(g * 16, 16)] = zvec
            return carry

        lax.fori_loop(0, ZR, zrow, 0)
        for j in range(RPT // ZR):
            pltpu.sync_copy(zero_v, acc_sh.at[pl.ds(sid * RPT + j * ZR, ZR)])

        @pl.when(sid == 0)
        def _zero_tail():
            pltpu.sync_copy(zero_v.at[pl.ds(0, TAIL)],
                            acc_sh.at[pl.ds(NS * RPT, TAIL)])

        plsc.subcore_barrier()

        ebase = (cid * NS + sid) * EPW

        def chunk(c, carry):
            base = ebase + c * K
            pltpu.sync_copy(sidx_hbm.at[pl.ds(base, K)], sidx_v)
            pltpu.sync_copy(didx_hbm.at[pl.ds(base, K)], didx_v)
            pltpu.async_copy(xsrc_hbm.at[sidx_v], rows_v, sem).wait()
            pltpu.sync_copy(p_hbm.at[pl.ds(base, K)], p_v)

            def edge(e, inner):
                for g in range(D // 16):
                    s = pl.ds(g * 16, 16)
                    p_v[e, s] = jnp.maximum(rows_v[e, s] + p_v[e, s], 0.0)
                return inner

            lax.fori_loop(0, K, edge, 0)
            pltpu.sync_copy(p_v, acc_sh.at[didx_v], add=True)
            return carry

        lax.fori_loop(0, NCHUNK, chunk, 0)
        plsc.subcore_barrier()
        pltpu.sync_copy(acc_sh.at[pl.ds(sid * RPT, RPT)],
                        out_hbm.at[cid, pl.ds(sid * RPT, RPT)])

        @pl.when(sid == 0)
        def _copy_tail():
            pltpu.sync_copy(acc_sh.at[pl.ds(NS * RPT, TAIL)],
                            out_hbm.at[cid, pl.ds(NS * RPT, TAIL)])

    return k(xsrc, sidx, didx, p)


def kernel(cons_embedding, vals_embedding, cons_embedding_0, vals_embedding_0,
           v2c_edge_index, c2v_edge_index, v2c_edge_attr, c2v_edge_attr,
           cons_batch, vals_batch, W_e1, W1, b1, W_e2, W2, b2):
    v2c_src = v2c_edge_index[0].astype(jnp.int32)
    v2c_dst = v2c_edge_index[1].astype(jnp.int32)
    c2v_src = c2v_edge_index[0].astype(jnp.int32)
    c2v_dst = c2v_edge_index[1].astype(jnp.int32)
    p1, p2 = _edge_bias(v2c_edge_attr, c2v_edge_attr, W_e1, W_e2)
    agg1 = _sc_agg(vals_embedding, v2c_src, v2c_dst, p1)
    cons_new = _update(cons_embedding, agg1, W1, b1.reshape(1, D))
    agg2 = _sc_agg(cons_new, c2v_src, c2v_dst, p2)
    vals_new = _update(vals_embedding, agg2, W2, b2.reshape(1, D))
    return (vals_new, cons_new)
